# baseline (speedup 1.0000x reference)
import functools

import jax
import jax.numpy as jnp
import numpy as np
from jax import lax
from jax.experimental import pallas as pl
from jax.experimental.pallas import tpu as pltpu
from jax.experimental.pallas import tpu_sc as plsc

_B, _T, _S, _V, _D = 4, 128, 2048, 32000, 1024
_VB = 1280
_NV = _V // _VB
_NW = 32
_RW = (_B * _T) // _NW


def _attn_body(x_ref, eq_ref, mask_ref, etgt_ref, wq_ref, wk_ref, wp_ref,
               bptr_ref, attn_ref, sw_ref):
    x = x_ref[0]
    eq = eq_ref[0].astype(jnp.bfloat16)
    qh = lax.dot_general(x.astype(jnp.bfloat16),
                         wq_ref[...].astype(jnp.bfloat16),
                         (((1,), (1,)), ((), ())),
                         preferred_element_type=jnp.float32)
    kh = lax.dot_general(eq, wk_ref[...].astype(jnp.bfloat16),
                         (((1,), (1,)), ((), ())),
                         preferred_element_type=jnp.float32)
    scores = lax.dot_general(qh.astype(jnp.bfloat16), kh.astype(jnp.bfloat16),
                             (((1,), (1,)), ((), ())),
                             preferred_element_type=jnp.float32)
    scores = scores * np.float32(1.0 / np.sqrt(_D))
    mask = mask_ref[0]
    scores = jnp.where(mask == 0.0, np.float32(-1e9), scores)
    m = jnp.max(scores, axis=1, keepdims=True)
    e = jnp.exp(scores - m)
    attn = e / jnp.sum(e, axis=1, keepdims=True)
    attn_ref[0] = attn
    tv = lax.dot_general(attn.astype(jnp.bfloat16), eq,
                         (((1,), (0,)), ((), ())),
                         preferred_element_type=jnp.float32)
    wp = wp_ref[...]
    etgt = etgt_ref[0]
    z = (jnp.sum(x * wp[0:1, :], axis=1, keepdims=True)
         + jnp.sum(tv * wp[1:2, :], axis=1, keepdims=True)
         + jnp.sum(etgt * wp[2:3, :], axis=1, keepdims=True)
         + bptr_ref[0, 0])
    sw_ref[0] = 1.0 / (1.0 + jnp.exp(-z))


def _attention(x, eq, mask, etgt, Wq, Wk, wp3, bptr):
    return pl.pallas_call(
        _attn_body,
        grid=(_B,),
        in_specs=[
            pl.BlockSpec((1, _T, _D), lambda b: (b, 0, 0)),
            pl.BlockSpec((1, _S, _D), lambda b: (b, 0, 0)),
            pl.BlockSpec((1, 1, _S), lambda b: (b, 0, 0)),
            pl.BlockSpec((1, _T, _D), lambda b: (b, 0, 0)),
            pl.BlockSpec((_D, _D), lambda b: (0, 0)),
            pl.BlockSpec((_D, _D), lambda b: (0, 0)),
            pl.BlockSpec((3, _D), lambda b: (0, 0)),
            pl.BlockSpec(memory_space=pltpu.SMEM),
        ],
        out_specs=[
            pl.BlockSpec((1, _T, _S), lambda b: (b, 0, 0)),
            pl.BlockSpec((1, _T, 1), lambda b: (b, 0, 0)),
        ],
        out_shape=[
            jax.ShapeDtypeStruct((_B, _T, _S), jnp.float32),
            jax.ShapeDtypeStruct((_B, _T, 1), jnp.float32),
        ],
    )(x, eq, mask, etgt, Wq, Wk, wp3, bptr)


def _sc_scatter_body(attn_hbm, query_hbm, pt_hbm, ids_v, row_v,
                     acc0, acc1, sem0, sem1):
    nc = 2
    wid = lax.axis_index("s") * nc + lax.axis_index("c")
    r0 = wid * _RW
    b = r0 // _T
    pltpu.sync_copy(query_hbm.at[b], ids_v)

    zeros16 = jnp.zeros((16,), jnp.float32)
    accs = (acc0, acc1)
    sems = (sem0, sem1)

    for acc in accs:
        def zero_body(i, c, acc=acc):
            base = i * 256
            for u in range(16):
                acc[pl.ds(base + u * 16, 16)] = zeros16
            return c
        lax.fori_loop(0, _V // 256, zero_body, 0)

    for r in range(_RW):
        acc = accs[r % 2]
        sem = sems[r % 2]
        row = r0 + r
        pltpu.sync_copy(attn_hbm.at[row], row_v)

        if r >= 2:
            pltpu.make_async_copy(acc, pt_hbm.at[row - 2], sem).wait()

            def rz_body(j, c, acc=acc):
                for u in range(8):
                    idx = ids_v[pl.ds((j * 8 + u) * 16, 16)]
                    plsc.store_scatter(acc, [idx], zeros16)
                return c
            lax.fori_loop(0, _S // 128, rz_body, 0)

        def scat_body(j, c, acc=acc):
            for u in range(8):
                base = (j * 8 + u) * 16
                idx = ids_v[pl.ds(base, 16)]
                val = row_v[pl.ds(base, 16)]
                plsc.addupdate_scatter(acc, [idx], val)
            return c
        lax.fori_loop(0, _S // 128, scat_body, 0)

        pltpu.async_copy(acc, pt_hbm.at[row], sem)

    pltpu.make_async_copy(acc0, pt_hbm.at[r0 + _RW - 2], sem0).wait()
    pltpu.make_async_copy(acc1, pt_hbm.at[r0 + _RW - 1], sem1).wait()


def _sc_scatter(attn2d, query_i32):
    mesh = plsc.VectorSubcoreMesh(core_axis_name="c", subcore_axis_name="s")
    f = pl.kernel(
        _sc_scatter_body,
        out_type=jax.ShapeDtypeStruct((_B * _T, _V), jnp.float32),
        mesh=mesh,
        scratch_types=[
            pltpu.VMEM((_S,), jnp.int32),
            pltpu.VMEM((_S,), jnp.float32),
            pltpu.VMEM((_V,), jnp.float32),
            pltpu.VMEM((_V,), jnp.float32),
            pltpu.SemaphoreType.DMA,
            pltpu.SemaphoreType.DMA,
        ],
        compiler_params=pltpu.CompilerParams(needs_layout_passes=False),
    )
    return f(attn2d, query_i32)


def _vocab_body(x_ref, vg_ref, pt_ref, sw_ref, out_ref, lb_scr, m_scr, l_scr):
    p = pl.program_id(0)
    k = pl.program_id(1)

    @pl.when(p == 0)
    def _phase0():
        X = x_ref[...].astype(jnp.bfloat16)
        G = vg_ref[...].astype(jnp.bfloat16)
        logits = lax.dot_general(X, G, (((1,), (1,)), ((), ())),
                                 preferred_element_type=jnp.float32)
        lb_scr[k] = logits.astype(jnp.bfloat16)
        bm = jnp.max(logits, axis=1, keepdims=True)

        @pl.when(k == 0)
        def _init():
            m_scr[...] = bm
            l_scr[...] = jnp.sum(jnp.exp(logits - bm), axis=1, keepdims=True)

        @pl.when(k > 0)
        def _upd():
            m_old = m_scr[...]
            m_new = jnp.maximum(m_old, bm)
            l_scr[...] = (l_scr[...] * jnp.exp(m_old - m_new)
                          + jnp.sum(jnp.exp(logits - m_new), axis=1,
                                    keepdims=True))
            m_scr[...] = m_new

    @pl.when(p == 1)
    def _phase1():
        logits = lb_scr[k].astype(jnp.float32)
        p_voc = jnp.exp(logits - m_scr[...]) / l_scr[...]
        sw = sw_ref[:, 0:1]
        out_ref[...] = jnp.log(sw * p_voc + (1.0 - sw) * pt_ref[...]
                               + np.float32(1e-30))


def _vocab_combine(x2d, vg, ptext, sw_b):
    bt = _B * _T
    return pl.pallas_call(
        _vocab_body,
        grid=(2, _NV),
        in_specs=[
            pl.BlockSpec((bt, _D), lambda p, k: (0, 0)),
            pl.BlockSpec((_VB, _D), lambda p, k: ((1 - p) * k, 0)),
            pl.BlockSpec((bt, _VB), lambda p, k: (0, p * k)),
            pl.BlockSpec((bt, 128), lambda p, k: (0, 0)),
        ],
        out_specs=pl.BlockSpec((bt, _VB), lambda p, k: (0, p * k)),
        out_shape=jax.ShapeDtypeStruct((bt, _V), jnp.float32),
        scratch_shapes=[
            pltpu.VMEM((_NV, bt, _VB), jnp.bfloat16),
            pltpu.VMEM((bt, 1), jnp.float32),
            pltpu.VMEM((bt, 1), jnp.float32),
        ],
    )(x2d, vg, ptext, sw_b)


@jax.jit
def kernel(x, query, encoded_query, query_mask, encoded_tgt, vocab_gen,
           Wq, Wk, W_ptr, b_ptr):
    wp3 = W_ptr.reshape(3, _D)
    bptr = b_ptr.reshape(1, 1)
    attn, sw = _attention(x, encoded_query, query_mask, encoded_tgt,
                          Wq, Wk, wp3, bptr)
    attn2d = attn.reshape(_B * _T, _S)
    query_i32 = query.astype(jnp.int32)
    ptext = _sc_scatter(attn2d, query_i32)
    x2d = x.reshape(_B * _T, _D)
    sw_b = jnp.broadcast_to(sw.reshape(_B * _T, 1), (_B * _T, 128))
    out = _vocab_combine(x2d, vocab_gen, ptext, sw_b)
    return out.reshape(_B, _T, _V)


# speedup vs baseline: 1.0684x; 1.0684x over previous
import functools

import jax
import jax.numpy as jnp
import numpy as np
from jax import lax
from jax.experimental import pallas as pl
from jax.experimental.pallas import tpu as pltpu
from jax.experimental.pallas import tpu_sc as plsc

_B, _T, _S, _V, _D = 4, 128, 2048, 32000, 1024
_VB = 1280
_NV = _V // _VB
_NW = 32
_RW = (_B * _T) // _NW


def _attn_body(x_ref, eq_ref, mask_ref, etgt_ref, wq_ref, wk_ref, wp_ref,
               bptr_ref, attn_ref, sw_ref):
    x = x_ref[0]
    eq = eq_ref[0].astype(jnp.bfloat16)
    qh = lax.dot_general(x.astype(jnp.bfloat16),
                         wq_ref[...].astype(jnp.bfloat16),
                         (((1,), (1,)), ((), ())),
                         preferred_element_type=jnp.float32)
    kh = lax.dot_general(eq, wk_ref[...].astype(jnp.bfloat16),
                         (((1,), (1,)), ((), ())),
                         preferred_element_type=jnp.float32)
    scores = lax.dot_general(qh.astype(jnp.bfloat16), kh.astype(jnp.bfloat16),
                             (((1,), (1,)), ((), ())),
                             preferred_element_type=jnp.float32)
    scores = scores * np.float32(1.0 / np.sqrt(_D))
    mask = mask_ref[0]
    scores = jnp.where(mask == 0.0, np.float32(-1e9), scores)
    m = jnp.max(scores, axis=1, keepdims=True)
    e = jnp.exp(scores - m)
    attn = e / jnp.sum(e, axis=1, keepdims=True)
    attn_ref[0] = attn
    tv = lax.dot_general(attn.astype(jnp.bfloat16), eq,
                         (((1,), (0,)), ((), ())),
                         preferred_element_type=jnp.float32)
    wp = wp_ref[...]
    etgt = etgt_ref[0]
    z = (jnp.sum(x * wp[0:1, :], axis=1, keepdims=True)
         + jnp.sum(tv * wp[1:2, :], axis=1, keepdims=True)
         + jnp.sum(etgt * wp[2:3, :], axis=1, keepdims=True)
         + bptr_ref[0, 0])
    sw_ref[0] = 1.0 / (1.0 + jnp.exp(-z))


def _attention(x, eq, mask, etgt, Wq, Wk, wp3, bptr):
    return pl.pallas_call(
        _attn_body,
        grid=(_B,),
        in_specs=[
            pl.BlockSpec((1, _T, _D), lambda b: (b, 0, 0)),
            pl.BlockSpec((1, _S, _D), lambda b: (b, 0, 0)),
            pl.BlockSpec((1, 1, _S), lambda b: (b, 0, 0)),
            pl.BlockSpec((1, _T, _D), lambda b: (b, 0, 0)),
            pl.BlockSpec((_D, _D), lambda b: (0, 0)),
            pl.BlockSpec((_D, _D), lambda b: (0, 0)),
            pl.BlockSpec((3, _D), lambda b: (0, 0)),
            pl.BlockSpec(memory_space=pltpu.SMEM),
        ],
        out_specs=[
            pl.BlockSpec((1, _T, _S), lambda b: (b, 0, 0)),
            pl.BlockSpec((1, _T, 1), lambda b: (b, 0, 0)),
        ],
        out_shape=[
            jax.ShapeDtypeStruct((_B, _T, _S), jnp.float32),
            jax.ShapeDtypeStruct((_B, _T, 1), jnp.float32),
        ],
    )(x, eq, mask, etgt, Wq, Wk, wp3, bptr)


def _sc_scatter_body(attn_hbm, query_hbm, pt_hbm, ids_v, row0, row1,
                     acc0, acc1, sem0, sem1, rsem0, rsem1):
    nc = 2
    wid = lax.axis_index("s") * nc + lax.axis_index("c")
    r0 = wid * _RW
    b = r0 // _T
    pltpu.sync_copy(query_hbm.at[b], ids_v)

    zeros16 = jnp.zeros((16,), jnp.float32)
    accs = (acc0, acc1)
    sems = (sem0, sem1)
    rows = (row0, row1)
    rsems = (rsem0, rsem1)

    pltpu.async_copy(attn_hbm.at[r0], row0, rsem0)

    for acc in accs:
        def zero_body(i, c, acc=acc):
            base = i * 256
            for u in range(16):
                acc[pl.ds(base + u * 16, 16)] = zeros16
            return c
        lax.fori_loop(0, _V // 256, zero_body, 0)

    for r in range(_RW):
        acc = accs[r % 2]
        sem = sems[r % 2]
        row_v = rows[r % 2]
        row = r0 + r

        if r >= 2:
            pltpu.make_async_copy(acc, pt_hbm.at[row - 2], sem).wait()

            def rz_body(j, c, acc=acc):
                for u in range(16):
                    idx = ids_v[pl.ds((j * 16 + u) * 16, 16)]
                    plsc.store_scatter(acc, [idx], zeros16)
                return c
            lax.fori_loop(0, _S // 256, rz_body, 0)

        pltpu.make_async_copy(attn_hbm.at[row], row_v, rsems[r % 2]).wait()
        if r + 1 < _RW:
            pltpu.async_copy(attn_hbm.at[row + 1], rows[(r + 1) % 2],
                             rsems[(r + 1) % 2])

        def scat_body(j, c, acc=acc, row_v=row_v):
            for u in range(16):
                base = (j * 16 + u) * 16
                idx = ids_v[pl.ds(base, 16)]
                val = row_v[pl.ds(base, 16)]
                plsc.addupdate_scatter(acc, [idx], val)
            return c
        lax.fori_loop(0, _S // 256, scat_body, 0)

        pltpu.async_copy(acc, pt_hbm.at[row], sem)

    pltpu.make_async_copy(acc0, pt_hbm.at[r0 + _RW - 2], sem0).wait()
    pltpu.make_async_copy(acc1, pt_hbm.at[r0 + _RW - 1], sem1).wait()


def _sc_scatter(attn2d, query_i32):
    mesh = plsc.VectorSubcoreMesh(core_axis_name="c", subcore_axis_name="s")
    f = pl.kernel(
        _sc_scatter_body,
        out_type=jax.ShapeDtypeStruct((_B * _T, _V), jnp.float32),
        mesh=mesh,
        scratch_types=[
            pltpu.VMEM((_S,), jnp.int32),
            pltpu.VMEM((_S,), jnp.float32),
            pltpu.VMEM((_S,), jnp.float32),
            pltpu.VMEM((_V,), jnp.float32),
            pltpu.VMEM((_V,), jnp.float32),
            pltpu.SemaphoreType.DMA,
            pltpu.SemaphoreType.DMA,
            pltpu.SemaphoreType.DMA,
            pltpu.SemaphoreType.DMA,
        ],
        compiler_params=pltpu.CompilerParams(needs_layout_passes=False),
    )
    return f(attn2d, query_i32)


def _stats_body(x_ref, vg_ref, m_ref, l_ref, lb_ref, m_scr, l_scr):
    k = pl.program_id(0)
    X = x_ref[...].astype(jnp.bfloat16)
    G = vg_ref[...].astype(jnp.bfloat16)
    logits = lax.dot_general(X, G, (((1,), (1,)), ((), ())),
                             preferred_element_type=jnp.float32)
    lb_ref[...] = logits.astype(jnp.bfloat16)
    bm = jnp.max(logits, axis=1, keepdims=True)

    @pl.when(k == 0)
    def _init():
        m_scr[...] = bm
        l_scr[...] = jnp.sum(jnp.exp(logits - bm), axis=1, keepdims=True)

    @pl.when(k > 0)
    def _upd():
        m_old = m_scr[...]
        m_new = jnp.maximum(m_old, bm)
        l_scr[...] = (l_scr[...] * jnp.exp(m_old - m_new)
                      + jnp.sum(jnp.exp(logits - m_new), axis=1,
                                keepdims=True))
        m_scr[...] = m_new

    m_ref[...] = m_scr[...]
    l_ref[...] = l_scr[...]


def _softmax_stats(x2d, vg):
    bt = _B * _T
    return pl.pallas_call(
        _stats_body,
        grid=(_NV,),
        in_specs=[
            pl.BlockSpec((bt, _D), lambda k: (0, 0)),
            pl.BlockSpec((_VB, _D), lambda k: (k, 0)),
        ],
        out_specs=[
            pl.BlockSpec((bt, 1), lambda k: (0, 0)),
            pl.BlockSpec((bt, 1), lambda k: (0, 0)),
            pl.BlockSpec((bt, _VB), lambda k: (0, k)),
        ],
        out_shape=[
            jax.ShapeDtypeStruct((bt, 1), jnp.float32),
            jax.ShapeDtypeStruct((bt, 1), jnp.float32),
            jax.ShapeDtypeStruct((bt, _V), jnp.bfloat16),
        ],
        scratch_shapes=[
            pltpu.VMEM((bt, 1), jnp.float32),
            pltpu.VMEM((bt, 1), jnp.float32),
        ],
    )(x2d, vg)


def _combine_body(lb_ref, pt_ref, sw_ref, m_ref, l_ref, out_ref):
    logits = lb_ref[...].astype(jnp.float32)
    p_voc = jnp.exp(logits - m_ref[...]) / l_ref[...]
    sw = sw_ref[:, 0:1]
    out_ref[...] = jnp.log(sw * p_voc + (1.0 - sw) * pt_ref[...]
                           + np.float32(1e-30))


def _combine(lb, ptext, sw_b, m, l):
    bt = _B * _T
    return pl.pallas_call(
        _combine_body,
        grid=(_NV,),
        in_specs=[
            pl.BlockSpec((bt, _VB), lambda k: (0, k)),
            pl.BlockSpec((bt, _VB), lambda k: (0, k)),
            pl.BlockSpec((bt, 128), lambda k: (0, 0)),
            pl.BlockSpec((bt, 1), lambda k: (0, 0)),
            pl.BlockSpec((bt, 1), lambda k: (0, 0)),
        ],
        out_specs=pl.BlockSpec((bt, _VB), lambda k: (0, k)),
        out_shape=jax.ShapeDtypeStruct((bt, _V), jnp.float32),
    )(lb, ptext, sw_b, m, l)


@jax.jit
def kernel(x, query, encoded_query, query_mask, encoded_tgt, vocab_gen,
           Wq, Wk, W_ptr, b_ptr):
    wp3 = W_ptr.reshape(3, _D)
    bptr = b_ptr.reshape(1, 1)
    attn, sw = _attention(x, encoded_query, query_mask, encoded_tgt,
                          Wq, Wk, wp3, bptr)
    attn2d = attn.reshape(_B * _T, _S)
    query_i32 = query.astype(jnp.int32)
    ptext = _sc_scatter(attn2d, query_i32)
    x2d = x.reshape(_B * _T, _D)
    sw_b = jnp.broadcast_to(sw.reshape(_B * _T, 1), (_B * _T, 128))
    m, l, lb = _softmax_stats(x2d, vocab_gen)
    out = _combine(lb, ptext, sw_b, m, l)
    return out.reshape(_B, _T, _V)


# speedup vs baseline: 1.1311x; 1.0587x over previous
import functools

import jax
import jax.numpy as jnp
import numpy as np
from jax import lax
from jax.experimental import pallas as pl
from jax.experimental.pallas import tpu as pltpu
from jax.experimental.pallas import tpu_sc as plsc

_B, _T, _S, _V, _D = 4, 128, 2048, 32000, 1024
_VB = 3200
_NV = _V // _VB
_NW = 32
_RW = (_B * _T) // _NW


def _attn_body(x_ref, eq_ref, mask_ref, etgt_ref, wq_ref, wk_ref, wp_ref,
               bptr_ref, attn_ref, sw_ref):
    x = x_ref[0]
    eq = eq_ref[0].astype(jnp.bfloat16)
    qh = lax.dot_general(x.astype(jnp.bfloat16),
                         wq_ref[...].astype(jnp.bfloat16),
                         (((1,), (1,)), ((), ())),
                         preferred_element_type=jnp.float32)
    kh = lax.dot_general(eq, wk_ref[...].astype(jnp.bfloat16),
                         (((1,), (1,)), ((), ())),
                         preferred_element_type=jnp.float32)
    scores = lax.dot_general(qh.astype(jnp.bfloat16), kh.astype(jnp.bfloat16),
                             (((1,), (1,)), ((), ())),
                             preferred_element_type=jnp.float32)
    scores = scores * np.float32(1.0 / np.sqrt(_D))
    mask = mask_ref[0]
    scores = jnp.where(mask == 0.0, np.float32(-1e9), scores)
    m = jnp.max(scores, axis=1, keepdims=True)
    e = jnp.exp(scores - m)
    attn = e / jnp.sum(e, axis=1, keepdims=True)
    attn_ref[0] = attn
    tv = lax.dot_general(attn.astype(jnp.bfloat16), eq,
                         (((1,), (0,)), ((), ())),
                         preferred_element_type=jnp.float32)
    wp = wp_ref[...]
    etgt = etgt_ref[0]
    z = (jnp.sum(x * wp[0:1, :], axis=1, keepdims=True)
         + jnp.sum(tv * wp[1:2, :], axis=1, keepdims=True)
         + jnp.sum(etgt * wp[2:3, :], axis=1, keepdims=True)
         + bptr_ref[0, 0])
    sw_ref[0] = 1.0 / (1.0 + jnp.exp(-z))


def _attention(x, eq, mask, etgt, Wq, Wk, wp3, bptr):
    return pl.pallas_call(
        _attn_body,
        grid=(_B,),
        in_specs=[
            pl.BlockSpec((1, _T, _D), lambda b: (b, 0, 0)),
            pl.BlockSpec((1, _S, _D), lambda b: (b, 0, 0)),
            pl.BlockSpec((1, 1, _S), lambda b: (b, 0, 0)),
            pl.BlockSpec((1, _T, _D), lambda b: (b, 0, 0)),
            pl.BlockSpec((_D, _D), lambda b: (0, 0)),
            pl.BlockSpec((_D, _D), lambda b: (0, 0)),
            pl.BlockSpec((3, _D), lambda b: (0, 0)),
            pl.BlockSpec(memory_space=pltpu.SMEM),
        ],
        out_specs=[
            pl.BlockSpec((1, _T, _S), lambda b: (b, 0, 0)),
            pl.BlockSpec((1, _T, 1), lambda b: (b, 0, 0)),
        ],
        out_shape=[
            jax.ShapeDtypeStruct((_B, _T, _S), jnp.float32),
            jax.ShapeDtypeStruct((_B, _T, 1), jnp.float32),
        ],
    )(x, eq, mask, etgt, Wq, Wk, wp3, bptr)


def _sc_scatter_body(attn_hbm, query_hbm, pt_hbm, ids_v, row0, row1,
                     acc0, acc1, sem0, sem1, rsem0, rsem1):
    nc = 2
    wid = lax.axis_index("s") * nc + lax.axis_index("c")
    r0 = wid * _RW
    b = r0 // _T
    pltpu.sync_copy(query_hbm.at[b], ids_v)

    zeros16 = jnp.zeros((16,), jnp.float32)
    accs = (acc0, acc1)
    sems = (sem0, sem1)
    rows = (row0, row1)
    rsems = (rsem0, rsem1)

    pltpu.async_copy(attn_hbm.at[r0], row0, rsem0)

    for acc in accs:
        def zero_body(i, c, acc=acc):
            base = i * 256
            for u in range(16):
                acc[pl.ds(base + u * 16, 16)] = zeros16
            return c
        lax.fori_loop(0, _V // 256, zero_body, 0)

    for r in range(_RW):
        acc = accs[r % 2]
        sem = sems[r % 2]
        row_v = rows[r % 2]
        row = r0 + r

        if r >= 2:
            pltpu.make_async_copy(acc, pt_hbm.at[row - 2], sem).wait()

            def rz_body(j, c, acc=acc):
                for u in range(16):
                    idx = ids_v[pl.ds((j * 16 + u) * 16, 16)]
                    plsc.store_scatter(acc, [idx], zeros16)
                return c
            lax.fori_loop(0, _S // 256, rz_body, 0)

        pltpu.make_async_copy(attn_hbm.at[row], row_v, rsems[r % 2]).wait()
        if r + 1 < _RW:
            pltpu.async_copy(attn_hbm.at[row + 1], rows[(r + 1) % 2],
                             rsems[(r + 1) % 2])

        def scat_body(j, c, acc=acc, row_v=row_v):
            for u in range(16):
                base = (j * 16 + u) * 16
                idx = ids_v[pl.ds(base, 16)]
                val = row_v[pl.ds(base, 16)]
                plsc.addupdate_scatter(acc, [idx], val)
            return c
        lax.fori_loop(0, _S // 256, scat_body, 0)

        pltpu.async_copy(acc, pt_hbm.at[row], sem)

    pltpu.make_async_copy(acc0, pt_hbm.at[r0 + _RW - 2], sem0).wait()
    pltpu.make_async_copy(acc1, pt_hbm.at[r0 + _RW - 1], sem1).wait()


def _sc_scatter(attn2d, query_i32):
    mesh = plsc.VectorSubcoreMesh(core_axis_name="c", subcore_axis_name="s")
    f = pl.kernel(
        _sc_scatter_body,
        out_type=jax.ShapeDtypeStruct((_B * _T, _V), jnp.float32),
        mesh=mesh,
        scratch_types=[
            pltpu.VMEM((_S,), jnp.int32),
            pltpu.VMEM((_S,), jnp.float32),
            pltpu.VMEM((_S,), jnp.float32),
            pltpu.VMEM((_V,), jnp.float32),
            pltpu.VMEM((_V,), jnp.float32),
            pltpu.SemaphoreType.DMA,
            pltpu.SemaphoreType.DMA,
            pltpu.SemaphoreType.DMA,
            pltpu.SemaphoreType.DMA,
        ],
        compiler_params=pltpu.CompilerParams(needs_layout_passes=False),
    )
    return f(attn2d, query_i32)


def _stats_body(x_ref, vg_ref, m_ref, l_ref, lb_ref, m_scr, l_scr):
    k = pl.program_id(0)
    X = x_ref[...].astype(jnp.bfloat16)
    G = vg_ref[...].astype(jnp.bfloat16)
    logits = lax.dot_general(X, G, (((1,), (1,)), ((), ())),
                             preferred_element_type=jnp.float32)
    lb_ref[...] = logits.astype(jnp.bfloat16)
    bm = jnp.max(logits, axis=1, keepdims=True)

    @pl.when(k == 0)
    def _init():
        m_scr[...] = bm
        l_scr[...] = jnp.sum(jnp.exp(logits - bm), axis=1, keepdims=True)

    @pl.when(k > 0)
    def _upd():
        m_old = m_scr[...]
        m_new = jnp.maximum(m_old, bm)
        l_scr[...] = (l_scr[...] * jnp.exp(m_old - m_new)
                      + jnp.sum(jnp.exp(logits - m_new), axis=1,
                                keepdims=True))
        m_scr[...] = m_new

    m_ref[...] = m_scr[...]
    l_ref[...] = l_scr[...]


def _softmax_stats(x2d, vg):
    bt = _B * _T
    return pl.pallas_call(
        _stats_body,
        grid=(_NV,),
        in_specs=[
            pl.BlockSpec((bt, _D), lambda k: (0, 0)),
            pl.BlockSpec((_VB, _D), lambda k: (k, 0)),
        ],
        out_specs=[
            pl.BlockSpec((bt, 1), lambda k: (0, 0)),
            pl.BlockSpec((bt, 1), lambda k: (0, 0)),
            pl.BlockSpec((bt, _VB), lambda k: (0, k)),
        ],
        out_shape=[
            jax.ShapeDtypeStruct((bt, 1), jnp.float32),
            jax.ShapeDtypeStruct((bt, 1), jnp.float32),
            jax.ShapeDtypeStruct((bt, _V), jnp.bfloat16),
        ],
        scratch_shapes=[
            pltpu.VMEM((bt, 1), jnp.float32),
            pltpu.VMEM((bt, 1), jnp.float32),
        ],
    )(x2d, vg)


def _combine_body(lb_ref, pt_ref, sw_ref, m_ref, l_ref, out_ref):
    logits = lb_ref[...].astype(jnp.float32)
    p_voc = jnp.exp(logits - m_ref[...]) / l_ref[...]
    sw = sw_ref[:, 0:1]
    out_ref[...] = jnp.log(sw * p_voc + (1.0 - sw) * pt_ref[...]
                           + np.float32(1e-30))


def _combine(lb, ptext, sw_b, m, l):
    bt = _B * _T
    return pl.pallas_call(
        _combine_body,
        grid=(_NV,),
        in_specs=[
            pl.BlockSpec((bt, _VB), lambda k: (0, k)),
            pl.BlockSpec((bt, _VB), lambda k: (0, k)),
            pl.BlockSpec((bt, 128), lambda k: (0, 0)),
            pl.BlockSpec((bt, 1), lambda k: (0, 0)),
            pl.BlockSpec((bt, 1), lambda k: (0, 0)),
        ],
        out_specs=pl.BlockSpec((bt, _VB), lambda k: (0, k)),
        out_shape=jax.ShapeDtypeStruct((bt, _V), jnp.float32),
    )(lb, ptext, sw_b, m, l)


@jax.jit
def kernel(x, query, encoded_query, query_mask, encoded_tgt, vocab_gen,
           Wq, Wk, W_ptr, b_ptr):
    wp3 = W_ptr.reshape(3, _D)
    bptr = b_ptr.reshape(1, 1)
    attn, sw = _attention(x, encoded_query, query_mask, encoded_tgt,
                          Wq, Wk, wp3, bptr)
    attn2d = attn.reshape(_B * _T, _S)
    query_i32 = query.astype(jnp.int32)
    ptext = _sc_scatter(attn2d, query_i32)
    x2d = x.reshape(_B * _T, _D)
    sw_b = jnp.broadcast_to(sw.reshape(_B * _T, 1), (_B * _T, 128))
    m, l, lb = _softmax_stats(x2d, vocab_gen)
    out = _combine(lb, ptext, sw_b, m, l)
    return out.reshape(_B, _T, _V)


# speedup vs baseline: 1.1393x; 1.0072x over previous
import functools

import jax
import jax.numpy as jnp
import numpy as np
from jax import lax
from jax.experimental import pallas as pl
from jax.experimental.pallas import tpu as pltpu
from jax.experimental.pallas import tpu_sc as plsc

_B, _T, _S, _V, _D = 4, 128, 2048, 32000, 1024
_VB = 3200
_NV = _V // _VB
_NW = 32
_RW = (_B * _T) // _NW


def _attn_body(x_ref, eq_ref, mask_ref, etgt_ref, wq_ref, wk_ref, wp_ref,
               bptr_ref, attn_ref, sw_ref):
    x = x_ref[0]
    eq = eq_ref[0].astype(jnp.bfloat16)
    qh = lax.dot_general(x.astype(jnp.bfloat16),
                         wq_ref[...].astype(jnp.bfloat16),
                         (((1,), (1,)), ((), ())),
                         preferred_element_type=jnp.float32)
    kh = lax.dot_general(eq, wk_ref[...].astype(jnp.bfloat16),
                         (((1,), (1,)), ((), ())),
                         preferred_element_type=jnp.float32)
    scores = lax.dot_general(qh.astype(jnp.bfloat16), kh.astype(jnp.bfloat16),
                             (((1,), (1,)), ((), ())),
                             preferred_element_type=jnp.float32)
    scores = scores * np.float32(1.0 / np.sqrt(_D))
    mask = mask_ref[0]
    scores = jnp.where(mask == 0.0, np.float32(-1e9), scores)
    m = jnp.max(scores, axis=1, keepdims=True)
    e = jnp.exp(scores - m)
    attn = e / jnp.sum(e, axis=1, keepdims=True)
    attn_ref[0] = attn
    tv = lax.dot_general(attn.astype(jnp.bfloat16), eq,
                         (((1,), (0,)), ((), ())),
                         preferred_element_type=jnp.float32)
    wp = wp_ref[...]
    etgt = etgt_ref[0]
    z = (jnp.sum(x * wp[0:1, :], axis=1, keepdims=True)
         + jnp.sum(tv * wp[1:2, :], axis=1, keepdims=True)
         + jnp.sum(etgt * wp[2:3, :], axis=1, keepdims=True)
         + bptr_ref[0, 0])
    sw_ref[0] = 1.0 / (1.0 + jnp.exp(-z))


def _attention(x, eq, mask, etgt, Wq, Wk, wp3, bptr):
    return pl.pallas_call(
        _attn_body,
        grid=(_B,),
        in_specs=[
            pl.BlockSpec((1, _T, _D), lambda b: (b, 0, 0)),
            pl.BlockSpec((1, _S, _D), lambda b: (b, 0, 0)),
            pl.BlockSpec((1, 1, _S), lambda b: (b, 0, 0)),
            pl.BlockSpec((1, _T, _D), lambda b: (b, 0, 0)),
            pl.BlockSpec((_D, _D), lambda b: (0, 0)),
            pl.BlockSpec((_D, _D), lambda b: (0, 0)),
            pl.BlockSpec((3, _D), lambda b: (0, 0)),
            pl.BlockSpec(memory_space=pltpu.SMEM),
        ],
        out_specs=[
            pl.BlockSpec((1, _T, _S), lambda b: (b, 0, 0)),
            pl.BlockSpec((1, _T, 1), lambda b: (b, 0, 0)),
        ],
        out_shape=[
            jax.ShapeDtypeStruct((_B, _T, _S), jnp.float32),
            jax.ShapeDtypeStruct((_B, _T, 1), jnp.float32),
        ],
    )(x, eq, mask, etgt, Wq, Wk, wp3, bptr)


def _sc_scatter_body(attn_hbm, query_hbm, pt_hbm, ids_v, row0, row1,
                     acc0, acc1, acc2, sem0, sem1, sem2, rsem0, rsem1):
    nc = 2
    wid = lax.axis_index("s") * nc + lax.axis_index("c")
    r0 = wid * _RW
    b = r0 // _T
    pltpu.sync_copy(query_hbm.at[b], ids_v)

    zeros16 = jnp.zeros((16,), jnp.float32)
    accs = (acc0, acc1, acc2)
    sems = (sem0, sem1, sem2)
    rows = (row0, row1)
    rsems = (rsem0, rsem1)
    nacc = len(accs)

    pltpu.async_copy(attn_hbm.at[r0], row0, rsem0)

    for acc in accs:
        def zero_body(i, c, acc=acc):
            base = i * 256
            for u in range(16):
                acc[pl.ds(base + u * 16, 16)] = zeros16
            return c
        lax.fori_loop(0, _V // 256, zero_body, 0)

    for r in range(_RW):
        acc = accs[r % nacc]
        sem = sems[r % nacc]
        row_v = rows[r % 2]
        row = r0 + r

        if r >= nacc:
            pltpu.make_async_copy(acc, pt_hbm.at[row - nacc], sem).wait()

            def rz_body(j, c, acc=acc):
                for u in range(16):
                    idx = ids_v[pl.ds((j * 16 + u) * 16, 16)]
                    plsc.store_scatter(acc, [idx], zeros16)
                return c
            lax.fori_loop(0, _S // 256, rz_body, 0)

        pltpu.make_async_copy(attn_hbm.at[row], row_v, rsems[r % 2]).wait()
        if r + 1 < _RW:
            pltpu.async_copy(attn_hbm.at[row + 1], rows[(r + 1) % 2],
                             rsems[(r + 1) % 2])

        def scat_body(j, c, acc=acc, row_v=row_v):
            for u in range(16):
                base = (j * 16 + u) * 16
                idx = ids_v[pl.ds(base, 16)]
                val = row_v[pl.ds(base, 16)]
                plsc.addupdate_scatter(acc, [idx], val)
            return c
        lax.fori_loop(0, _S // 256, scat_body, 0)

        pltpu.async_copy(acc, pt_hbm.at[row], sem)

    for d in range(nacc):
        r = _RW - nacc + d
        pltpu.make_async_copy(accs[r % nacc], pt_hbm.at[r0 + r],
                              sems[r % nacc]).wait()


def _sc_scatter(attn2d, query_i32):
    mesh = plsc.VectorSubcoreMesh(core_axis_name="c", subcore_axis_name="s")
    f = pl.kernel(
        _sc_scatter_body,
        out_type=jax.ShapeDtypeStruct((_B * _T, _V), jnp.float32),
        mesh=mesh,
        scratch_types=[
            pltpu.VMEM((_S,), jnp.int32),
            pltpu.VMEM((_S,), jnp.float32),
            pltpu.VMEM((_S,), jnp.float32),
            pltpu.VMEM((_V,), jnp.float32),
            pltpu.VMEM((_V,), jnp.float32),
            pltpu.VMEM((_V,), jnp.float32),
            pltpu.SemaphoreType.DMA,
            pltpu.SemaphoreType.DMA,
            pltpu.SemaphoreType.DMA,
            pltpu.SemaphoreType.DMA,
            pltpu.SemaphoreType.DMA,
        ],
        compiler_params=pltpu.CompilerParams(needs_layout_passes=False),
    )
    return f(attn2d, query_i32)


def _stats_body(x_ref, vg_ref, m_ref, l_ref, lb_ref, m_scr, l_scr):
    k = pl.program_id(0)
    X = x_ref[...].astype(jnp.bfloat16)
    G = vg_ref[...].astype(jnp.bfloat16)
    logits = lax.dot_general(X, G, (((1,), (1,)), ((), ())),
                             preferred_element_type=jnp.float32)
    lb_ref[...] = logits.astype(jnp.bfloat16)
    bm = jnp.max(logits, axis=1, keepdims=True)

    @pl.when(k == 0)
    def _init():
        m_scr[...] = bm
        l_scr[...] = jnp.sum(jnp.exp(logits - bm), axis=1, keepdims=True)

    @pl.when(k > 0)
    def _upd():
        m_old = m_scr[...]
        m_new = jnp.maximum(m_old, bm)
        l_scr[...] = (l_scr[...] * jnp.exp(m_old - m_new)
                      + jnp.sum(jnp.exp(logits - m_new), axis=1,
                                keepdims=True))
        m_scr[...] = m_new

    m_ref[...] = m_scr[...]
    l_ref[...] = l_scr[...]


def _softmax_stats(x2d, vg):
    bt = _B * _T
    return pl.pallas_call(
        _stats_body,
        grid=(_NV,),
        in_specs=[
            pl.BlockSpec((bt, _D), lambda k: (0, 0)),
            pl.BlockSpec((_VB, _D), lambda k: (k, 0)),
        ],
        out_specs=[
            pl.BlockSpec((bt, 1), lambda k: (0, 0)),
            pl.BlockSpec((bt, 1), lambda k: (0, 0)),
            pl.BlockSpec((bt, _VB), lambda k: (0, k)),
        ],
        out_shape=[
            jax.ShapeDtypeStruct((bt, 1), jnp.float32),
            jax.ShapeDtypeStruct((bt, 1), jnp.float32),
            jax.ShapeDtypeStruct((bt, _V), jnp.bfloat16),
        ],
        scratch_shapes=[
            pltpu.VMEM((bt, 1), jnp.float32),
            pltpu.VMEM((bt, 1), jnp.float32),
        ],
    )(x2d, vg)


def _combine_body(lb_ref, pt_ref, sw_ref, m_ref, l_ref, out_ref):
    logits = lb_ref[...].astype(jnp.float32)
    p_voc = jnp.exp(logits - m_ref[...]) / l_ref[...]
    sw = sw_ref[:, 0:1]
    out_ref[...] = jnp.log(sw * p_voc + (1.0 - sw) * pt_ref[...]
                           + np.float32(1e-30))


def _combine(lb, ptext, sw_b, m, l):
    bt = _B * _T
    return pl.pallas_call(
        _combine_body,
        grid=(_NV,),
        in_specs=[
            pl.BlockSpec((bt, _VB), lambda k: (0, k)),
            pl.BlockSpec((bt, _VB), lambda k: (0, k)),
            pl.BlockSpec((bt, 128), lambda k: (0, 0)),
            pl.BlockSpec((bt, 1), lambda k: (0, 0)),
            pl.BlockSpec((bt, 1), lambda k: (0, 0)),
        ],
        out_specs=pl.BlockSpec((bt, _VB), lambda k: (0, k)),
        out_shape=jax.ShapeDtypeStruct((bt, _V), jnp.float32),
    )(lb, ptext, sw_b, m, l)


@jax.jit
def kernel(x, query, encoded_query, query_mask, encoded_tgt, vocab_gen,
           Wq, Wk, W_ptr, b_ptr):
    wp3 = W_ptr.reshape(3, _D)
    bptr = b_ptr.reshape(1, 1)
    attn, sw = _attention(x, encoded_query, query_mask, encoded_tgt,
                          Wq, Wk, wp3, bptr)
    attn2d = attn.reshape(_B * _T, _S)
    query_i32 = query.astype(jnp.int32)
    x2d = x.reshape(_B * _T, _D)
    ptext = _sc_scatter(attn2d, query_i32)
    m, l, lb = _softmax_stats(x2d, vocab_gen)
    sw_b = jnp.broadcast_to(sw.reshape(_B * _T, 1), (_B * _T, 128))
    out = _combine(lb, ptext, sw_b, m, l)
    return out.reshape(_B, _T, _V)
